# baseline (speedup 1.0000x reference)
import jax
import jax.numpy as jnp
from jax.experimental import pallas as pl

N = 50000
G = 1024


def _gcn_conv(x, edge_index, W, b, n):
    src = edge_index[0]
    dst = edge_index[1]
    loop = jnp.arange(n, dtype=src.dtype)
    src = jnp.concatenate([src, loop])
    dst = jnp.concatenate([dst, loop])
    deg = jnp.zeros((n,), jnp.float32).at[dst].add(1.0)
    dinv = jnp.where(deg > 0, jax.lax.rsqrt(jnp.maximum(deg, 1e-12)), 0.0)
    norm = dinv[src] * dinv[dst]
    h = x @ W
    msg = jnp.take(h, src, axis=0) * norm[:, None]
    out = jnp.zeros((n, h.shape[1]), jnp.float32).at[dst].add(msg)
    return out + b


def _mean_pool(z, batch, g):
    sums = jax.ops.segment_sum(z, batch, num_segments=g)
    cnt = jax.ops.segment_sum(jnp.ones((z.shape[0],), jnp.float32), batch, num_segments=g)
    return sums / jnp.maximum(cnt, 1.0)[:, None]


def _id_kernel(x_ref, o_ref):
    o_ref[...] = x_ref[...]


def kernel(xA, edge_indexA, batchA, xB, edge_indexB, batchB, W1, b1, Wmu, bmu, Wstd, bstd, R1, rb1, R2, rb2, R3, rb3):
    def encode(x, ei):
        n = x.shape[0]
        h = jax.nn.relu(_gcn_conv(x, ei, W1, b1, n))
        mu = _gcn_conv(h, ei, Wmu, bmu, n)
        return mu

    zA = _mean_pool(encode(xA, edge_indexA), batchA, G)
    zB = _mean_pool(encode(xB, edge_indexB), batchB, G)
    h = jnp.concatenate([zA, zB], axis=1)
    h = pl.pallas_call(
        _id_kernel, out_shape=jax.ShapeDtypeStruct(h.shape, h.dtype))(h)
    h = jax.nn.relu(h @ R1 + rb1)
    h = jax.nn.relu(h @ R2 + rb2)
    return h @ R3 + rb3


# speedup vs baseline: 15.4646x; 15.4646x over previous
import functools

import jax
import jax.numpy as jnp
from jax import lax
from jax.experimental import pallas as pl
from jax.experimental.pallas import tpu as pltpu
from jax.experimental.pallas import tpu_sc as plsc

N = 50000
E = 800000
D = 58
DP = 64
HID = 128
LAT = 32
G = 1024

NC = 2
NS = 16
SUB = 128
ERP = 6400
EPT = ERP // NS
NACC = 51200
RPT = NACC // NS
GACC = 1040
GPT = GACC // NS
NB = 400
NBPT = NB // NS
NPAD = NB * SUB
NBATCH = 2
CHUNK = 50

_mesh = plsc.VectorSubcoreMesh(
    core_axis_name="c", subcore_axis_name="s", num_cores=NC, num_subcores=NS)
_sc_params = pltpu.CompilerParams(use_tc_tiling_on_sc=False)


def _zero_vmem(ref, n_f32):
    z = jnp.zeros((16,), jnp.float32)
    flat = n_f32 // 16
    if ref.ndim == 1:
        for i in range(flat):
            ref[pl.ds(i * 16, 16)] = z
    else:
        rows, cols = ref.shape
        per_row = cols // 16
        for r in range(rows):
            for j in range(per_row):
                ref[r, pl.ds(j * 16, 16)] = z


def _deg_body(dst2, deg_out, acc, idx, ones_v, zb, wb, sem):
    c = lax.axis_index("c")
    s = lax.axis_index("s")
    base = pl.multiple_of(s * RPT, 8)
    _zero_vmem(zb, RPT)
    for i in range(8):
        ones_v[pl.ds(i * 16, 16)] = jnp.ones((16,), jnp.float32)
    pltpu.sync_copy(zb, acc.at[pl.ds(base, RPT)])
    plsc.subcore_barrier()
    pltpu.sync_copy(dst2.at[c, pl.ds(s * EPT, EPT)], idx)

    def batch(b, _):
        descs = [
            pltpu.async_copy(ones_v, acc.at[idx.at[b * NBATCH + j]], sem,
                             add=True)
            for j in range(NBATCH)
        ]
        for d in descs:
            d.wait()
        return _

    lax.fori_loop(0, EPT // NBATCH, batch, None)
    plsc.subcore_barrier()
    pltpu.sync_copy(acc.at[pl.ds(base, RPT)], wb)
    pltpu.sync_copy(wb, deg_out.at[c, pl.ds(base, RPT)])


_deg_kernel = functools.partial(
    pl.kernel,
    out_type=jax.ShapeDtypeStruct((NC, NACC), jnp.float32),
    mesh=_mesh,
    compiler_params=_sc_params,
    scratch_types=[
        pltpu.VMEM_SHARED((NACC,), jnp.float32),
        pltpu.VMEM((EPT, SUB), jnp.int32),
        pltpu.VMEM((SUB,), jnp.float32),
        pltpu.VMEM((RPT,), jnp.float32),
        pltpu.VMEM((RPT,), jnp.float32),
        pltpu.SemaphoreType.DMA,
    ],
)(_deg_body)


def _edge_pass(tab, src2, dst2, out, acc, idx_s, idx_d, rbuf, zrow,
               semg, sems, s):
    base = pl.multiple_of(s * RPT, 8)
    _zero_vmem(zrow, SUB * 32)
    for i in range(RPT // SUB):
        pltpu.sync_copy(zrow, acc.at[pl.ds(base + i * SUB, SUB)])
    plsc.subcore_barrier()
    for hchunk in range(EPT // CHUNK):
        row0 = s * EPT + hchunk * CHUNK
        pltpu.sync_copy(src2.at[pl.ds(row0, CHUNK)], idx_s)
        pltpu.sync_copy(dst2.at[pl.ds(row0, CHUNK)], idx_d)

        def batch(b, _):
            r = b * NBATCH
            gd = [
                pltpu.async_copy(tab.at[idx_s.at[r + j]], rbuf.at[j], semg)
                for j in range(NBATCH)
            ]
            for d in gd:
                d.wait()
            sd = [
                pltpu.async_copy(rbuf.at[j], acc.at[idx_d.at[r + j]], sems,
                                 add=True)
                for j in range(NBATCH)
            ]
            for d in sd:
                d.wait()
            return _

        lax.fori_loop(0, CHUNK // NBATCH, batch, None)
    plsc.subcore_barrier()
    for i in range(RPT // SUB):
        pltpu.sync_copy(acc.at[pl.ds(base + i * SUB, SUB)], zrow)
        pltpu.sync_copy(zrow, out.at[pl.ds(base + i * SUB, SUB)])


def _conv1_body(tabs, src2, dst2, s1, acc, idx_s, idx_d, rbuf, zrow,
                semg, sems):
    c = lax.axis_index("c")
    s = lax.axis_index("s")
    for g in range(2):
        _edge_pass(tabs.at[g, c], src2.at[g], dst2.at[g], s1.at[g, c],
                   acc, idx_s, idx_d, rbuf, zrow, semg, sems, s)
        if g == 0:
            plsc.subcore_barrier()


_conv1_kernel = functools.partial(
    pl.kernel,
    out_type=jax.ShapeDtypeStruct((2, NC, NACC, 32), jnp.float32),
    mesh=_mesh,
    compiler_params=_sc_params,
    scratch_types=[
        pltpu.VMEM_SHARED((NACC, 32), jnp.float32),
        pltpu.VMEM((CHUNK, SUB), jnp.int32),
        pltpu.VMEM((CHUNK, SUB), jnp.int32),
        pltpu.VMEM((NBATCH, SUB, 32), jnp.float32),
        pltpu.VMEM((SUB, 32), jnp.float32),
        pltpu.SemaphoreType.DMA,
        pltpu.SemaphoreType.DMA,
    ],
)(_conv1_body)


def _conv2_body(tabs, src2, dst2, s2, acc, idx_s, idx_d, rbuf, zrow,
                semg, sems):
    c = lax.axis_index("c")
    s = lax.axis_index("s")
    _edge_pass(tabs.at[c], src2.at[c], dst2.at[c], s2.at[c],
               acc, idx_s, idx_d, rbuf, zrow, semg, sems, s)


_conv2_kernel = functools.partial(
    pl.kernel,
    out_type=jax.ShapeDtypeStruct((NC, NACC, 32), jnp.float32),
    mesh=_mesh,
    compiler_params=_sc_params,
    scratch_types=[
        pltpu.VMEM_SHARED((NACC, 32), jnp.float32),
        pltpu.VMEM((CHUNK, SUB), jnp.int32),
        pltpu.VMEM((CHUNK, SUB), jnp.int32),
        pltpu.VMEM((NBATCH, SUB, 32), jnp.float32),
        pltpu.VMEM((SUB, 32), jnp.float32),
        pltpu.SemaphoreType.DMA,
        pltpu.SemaphoreType.DMA,
    ],
)(_conv2_body)


def _pool_body(vext, bat2, pool_out, acc, idx, vbuf, zrow, wb, semg, sems):
    c = lax.axis_index("c")
    s = lax.axis_index("s")
    _zero_vmem(zrow, GPT * 48)
    pltpu.sync_copy(zrow, acc.at[pl.ds(s * GPT, GPT)])
    plsc.subcore_barrier()
    pltpu.sync_copy(bat2.at[c, pl.ds(s * NBPT, NBPT)], idx)

    def batch(b, _):
        r = b * 5
        gd = [
            pltpu.async_copy(
                vext.at[c, pl.ds((s * NBPT + r + j) * SUB, SUB)],
                vbuf.at[j], semg)
            for j in range(5)
        ]
        for d in gd:
            d.wait()
        sd = [
            pltpu.async_copy(vbuf.at[j], acc.at[idx.at[r + j]], sems,
                             add=True)
            for j in range(5)
        ]
        for d in sd:
            d.wait()
        return _

    lax.fori_loop(0, NBPT // 5, batch, None)
    plsc.subcore_barrier()
    pltpu.sync_copy(acc.at[pl.ds(s * GPT, GPT)], wb)
    pltpu.sync_copy(wb, pool_out.at[c, pl.ds(s * GPT, GPT)])


_pool_kernel = functools.partial(
    pl.kernel,
    out_type=jax.ShapeDtypeStruct((NC, GACC, 48), jnp.float32),
    mesh=_mesh,
    compiler_params=_sc_params,
    scratch_types=[
        pltpu.VMEM_SHARED((GACC, 48), jnp.float32),
        pltpu.VMEM((NBPT, SUB), jnp.int32),
        pltpu.VMEM((5, SUB, 48), jnp.float32),
        pltpu.VMEM((GPT, 48), jnp.float32),
        pltpu.VMEM((GPT, 48), jnp.float32),
        pltpu.SemaphoreType.DMA,
        pltpu.SemaphoreType.DMA,
    ],
)(_pool_body)


def _mid_body(y_ref, w1_ref, b1_ref, wmu_ref, u_ref):
    y = y_ref[0]
    h = jnp.maximum(
        jnp.dot(y, w1_ref[...], preferred_element_type=jnp.float32)
        + b1_ref[...], 0.0)
    u_ref[0] = jnp.dot(h, wmu_ref[...], preferred_element_type=jnp.float32)


def _mid_call(y1pre, w1p, b1, wmu):
    blk = 2000
    return pl.pallas_call(
        _mid_body,
        grid=(2, N // blk),
        in_specs=[
            pl.BlockSpec((1, blk, DP), lambda g, i: (g, i, 0)),
            pl.BlockSpec((DP, HID), lambda g, i: (0, 0)),
            pl.BlockSpec((1, HID), lambda g, i: (0, 0)),
            pl.BlockSpec((HID, LAT), lambda g, i: (0, 0)),
        ],
        out_specs=pl.BlockSpec((1, blk, LAT), lambda g, i: (g, i, 0)),
        out_shape=jax.ShapeDtypeStruct((2, N, LAT), jnp.float32),
    )(y1pre, w1p, b1.reshape(1, HID), wmu)


def _head_body(h_ref, r1_ref, rb1_ref, r2_ref, rb2_ref, r3_ref, rb3_ref,
               o_ref):
    h = jnp.maximum(
        jnp.dot(h_ref[...], r1_ref[...], preferred_element_type=jnp.float32)
        + rb1_ref[...], 0.0)
    h = jnp.maximum(
        jnp.dot(h, r2_ref[...], preferred_element_type=jnp.float32)
        + rb2_ref[...], 0.0)
    o_ref[...] = (
        jnp.dot(h, r3_ref[...], preferred_element_type=jnp.float32)
        + rb3_ref[...])


def _head_call(hcat, R1, rb1, R2, rb2, R3, rb3):
    return pl.pallas_call(
        _head_body,
        out_shape=jax.ShapeDtypeStruct((G, 2), jnp.float32),
    )(hcat, R1, rb1.reshape(1, -1), R2, rb2.reshape(1, -1), R3,
      rb3.reshape(1, -1))


def _pad_rows(a, rows, val):
    return jnp.pad(a, ((0, rows - a.shape[0]),), constant_values=val)


def kernel(xA, edge_indexA, batchA, xB, edge_indexB, batchB, W1, b1, Wmu, bmu,
           Wstd, bstd, R1, rb1, R2, rb2, R3, rb3):
    def idx2d(v, pad_val):
        return _pad_rows(v, ERP * SUB, pad_val).reshape(ERP, SUB)

    src2 = jnp.stack([idx2d(edge_indexA[0], 0), idx2d(edge_indexB[0], 0)])
    dst2 = jnp.stack([idx2d(edge_indexA[1], N), idx2d(edge_indexB[1], N)])
    bat2 = jnp.stack([
        _pad_rows(batchA, NPAD, G).reshape(NB, SUB),
        _pad_rows(batchB, NPAD, G).reshape(NB, SUB),
    ])

    degs = _deg_kernel(dst2)[:, :N]
    dinv = lax.rsqrt(degs + 1.0)

    x = jnp.stack([xA, xB])
    xp = jnp.pad(x * dinv[:, :, None], ((0, 0), (0, 0), (0, DP - D)))
    tabs1 = jnp.stack([xp[:, :, :32], xp[:, :, 32:]], axis=1)
    s1 = _conv1_kernel(tabs1, src2, dst2)
    s1f = jnp.concatenate([s1[:, 0, :N], s1[:, 1, :N]], axis=-1)
    y1pre = dinv[:, :, None] * (s1f + xp)

    w1p = jnp.pad(W1, ((0, DP - D), (0, 0)))
    u = _mid_call(y1pre, w1p, b1, Wmu)
    up = dinv[:, :, None] * u

    s2 = _conv2_kernel(up, src2, dst2)
    v = dinv[:, :, None] * (s2[:, :N] + up)
    vext = jnp.concatenate([
        jnp.pad(v, ((0, 0), (0, NPAD - N), (0, 0))),
        jnp.pad(jnp.ones((2, N, 1), jnp.float32),
                ((0, 0), (0, NPAD - N), (0, 0))),
        jnp.zeros((2, NPAD, 15), jnp.float32),
    ], axis=-1)
    pooled = _pool_kernel(vext, bat2)
    sums = pooled[:, :G, :LAT]
    cnt = pooled[:, :G, LAT]
    z = (sums + cnt[:, :, None] * bmu) / jnp.maximum(cnt, 1.0)[:, :, None]

    hcat = jnp.concatenate([z[0], z[1]], axis=1)
    return _head_call(hcat, R1, rb1, R2, rb2, R3, rb3)


# speedup vs baseline: 16.7692x; 1.0844x over previous
import functools

import jax
import jax.numpy as jnp
from jax import lax
from jax.experimental import pallas as pl
from jax.experimental.pallas import tpu as pltpu
from jax.experimental.pallas import tpu_sc as plsc

N = 50000
E = 800000
D = 58
DP = 64
HID = 128
LAT = 32
G = 1024

NC = 2
NS = 16
EP = 819200
EPT = EP // NS
NACC = 51200
RPT = NACC // NS
GACC = 1040
GPT = GACC // NS
NPAD = 51200
NPT = NPAD // NS

LS = 256
NB2 = 2
ECH = 2048
DCH = 2048
PCH = 640

_mesh = plsc.VectorSubcoreMesh(
    core_axis_name="c", subcore_axis_name="s", num_cores=NC, num_subcores=NS)
_sc_params = pltpu.CompilerParams(use_tc_tiling_on_sc=False)


def _zero_vmem(ref, n_f32):
    z = jnp.zeros((16,), jnp.float32)
    if ref.ndim == 1:
        for i in range(n_f32 // 16):
            ref[pl.ds(i * 16, 16)] = z
    else:
        rows, cols = ref.shape
        for r in range(rows):
            for j in range(cols // 16):
                ref[r, pl.ds(j * 16, 16)] = z


def _deg_body(dst1, deg_out, acc, idx, ones_v, zb, sem):
    c = lax.axis_index("c")
    s = lax.axis_index("s")
    base = pl.multiple_of(s * RPT, 8)
    _zero_vmem(zb, RPT)
    for i in range(DCH // 16):
        ones_v[pl.ds(i * 16, 16)] = jnp.ones((16,), jnp.float32)
    pltpu.sync_copy(zb, acc.at[pl.ds(base, RPT)])
    plsc.subcore_barrier()
    pltpu.sync_copy(dst1.at[c, pl.ds(s * EPT, EPT)], idx)
    for b in range(EPT // DCH):
        pltpu.async_copy(ones_v, acc.at[idx.at[pl.ds(b * DCH, DCH)]], sem,
                         add=True).wait()
    plsc.subcore_barrier()
    pltpu.sync_copy(acc.at[pl.ds(base, RPT)], zb)
    pltpu.sync_copy(zb, deg_out.at[c, pl.ds(base, RPT)])


_deg_kernel = functools.partial(
    pl.kernel,
    out_type=jax.ShapeDtypeStruct((NC, NACC), jnp.float32),
    mesh=_mesh,
    compiler_params=_sc_params,
    scratch_types=[
        pltpu.VMEM_SHARED((NACC,), jnp.float32),
        pltpu.VMEM((EPT,), jnp.int32),
        pltpu.VMEM((DCH,), jnp.float32),
        pltpu.VMEM((RPT,), jnp.float32),
        pltpu.SemaphoreType.DMA,
    ],
)(_deg_body)


def _edge_pass(tab, src1, dst1, out, acc, idx_s, idx_d, rbuf, zrow,
               semg, sems, s):
    base = pl.multiple_of(s * RPT, 8)
    _zero_vmem(zrow, 128 * 32)
    for i in range(RPT // 128):
        pltpu.sync_copy(zrow, acc.at[pl.ds(base + i * 128, 128)])
    plsc.subcore_barrier()
    for step in range(EPT // ECH):
        e0 = s * EPT + step * ECH
        pltpu.sync_copy(src1.at[pl.ds(e0, ECH)], idx_s)
        pltpu.sync_copy(dst1.at[pl.ds(e0, ECH)], idx_d)

        def batch(b, _):
            r = pl.multiple_of(b * (NB2 * LS), LS)
            gd = [
                pltpu.async_copy(tab.at[idx_s.at[pl.ds(r + j * LS, LS)]],
                                 rbuf.at[j], semg)
                for j in range(NB2)
            ]
            for d in gd:
                d.wait()
            sd = [
                pltpu.async_copy(rbuf.at[j],
                                 acc.at[idx_d.at[pl.ds(r + j * LS, LS)]],
                                 sems, add=True)
                for j in range(NB2)
            ]
            for d in sd:
                d.wait()
            return _

        lax.fori_loop(0, ECH // (NB2 * LS), batch, None)
    plsc.subcore_barrier()
    for i in range(RPT // 128):
        pltpu.sync_copy(acc.at[pl.ds(base + i * 128, 128)], zrow)
        pltpu.sync_copy(zrow, out.at[pl.ds(base + i * 128, 128)])


_conv_scratch = [
    pltpu.VMEM_SHARED((NACC, 32), jnp.float32),
    pltpu.VMEM((ECH,), jnp.int32),
    pltpu.VMEM((ECH,), jnp.int32),
    pltpu.VMEM((NB2, LS, 32), jnp.float32),
    pltpu.VMEM((128, 32), jnp.float32),
    pltpu.SemaphoreType.DMA,
    pltpu.SemaphoreType.DMA,
]


def _conv1_body(tabs, src1, dst1, s1, acc, idx_s, idx_d, rbuf, zrow,
                semg, sems):
    c = lax.axis_index("c")
    s = lax.axis_index("s")
    for g in range(2):
        _edge_pass(tabs.at[g, c], src1.at[g], dst1.at[g], s1.at[g, c],
                   acc, idx_s, idx_d, rbuf, zrow, semg, sems, s)
        if g == 0:
            plsc.subcore_barrier()


_conv1_kernel = functools.partial(
    pl.kernel,
    out_type=jax.ShapeDtypeStruct((2, NC, NACC, 32), jnp.float32),
    mesh=_mesh,
    compiler_params=_sc_params,
    scratch_types=_conv_scratch,
)(_conv1_body)


def _conv2_body(tabs, src1, dst1, s2, acc, idx_s, idx_d, rbuf, zrow,
                semg, sems):
    c = lax.axis_index("c")
    s = lax.axis_index("s")
    _edge_pass(tabs.at[c], src1.at[c], dst1.at[c], s2.at[c],
               acc, idx_s, idx_d, rbuf, zrow, semg, sems, s)


_conv2_kernel = functools.partial(
    pl.kernel,
    out_type=jax.ShapeDtypeStruct((NC, NACC, 32), jnp.float32),
    mesh=_mesh,
    compiler_params=_sc_params,
    scratch_types=_conv_scratch,
)(_conv2_body)


def _pool_body(vext, bat1, pool_out, acc, idx, vbuf, zrow, semg, sems):
    c = lax.axis_index("c")
    s = lax.axis_index("s")
    _zero_vmem(zrow, GPT * 48)
    pltpu.sync_copy(zrow, acc.at[pl.ds(s * GPT, GPT)])
    plsc.subcore_barrier()
    pltpu.sync_copy(bat1.at[c, pl.ds(s * NPT, NPT)], idx)
    for b in range(NPT // PCH):
        j = b % 2
        pltpu.async_copy(vext.at[c, pl.ds(s * NPT + b * PCH, PCH)],
                         vbuf.at[j], semg).wait()
        pltpu.async_copy(vbuf.at[j], acc.at[idx.at[pl.ds(b * PCH, PCH)]],
                         sems, add=True).wait()
    plsc.subcore_barrier()
    pltpu.sync_copy(acc.at[pl.ds(s * GPT, GPT)], zrow)
    pltpu.sync_copy(zrow, pool_out.at[c, pl.ds(s * GPT, GPT)])


_pool_kernel = functools.partial(
    pl.kernel,
    out_type=jax.ShapeDtypeStruct((NC, GACC, 48), jnp.float32),
    mesh=_mesh,
    compiler_params=_sc_params,
    scratch_types=[
        pltpu.VMEM_SHARED((GACC, 48), jnp.float32),
        pltpu.VMEM((NPT,), jnp.int32),
        pltpu.VMEM((2, PCH, 48), jnp.float32),
        pltpu.VMEM((GPT, 48), jnp.float32),
        pltpu.SemaphoreType.DMA,
        pltpu.SemaphoreType.DMA,
    ],
)(_pool_body)


def _mid_body(y_ref, w1_ref, b1_ref, wmu_ref, u_ref):
    y = y_ref[0]
    h = jnp.maximum(
        jnp.dot(y, w1_ref[...], preferred_element_type=jnp.float32)
        + b1_ref[...], 0.0)
    u_ref[0] = jnp.dot(h, wmu_ref[...], preferred_element_type=jnp.float32)


def _mid_call(y1pre, w1p, b1, wmu):
    blk = 2000
    return pl.pallas_call(
        _mid_body,
        grid=(2, N // blk),
        in_specs=[
            pl.BlockSpec((1, blk, DP), lambda g, i: (g, i, 0)),
            pl.BlockSpec((DP, HID), lambda g, i: (0, 0)),
            pl.BlockSpec((1, HID), lambda g, i: (0, 0)),
            pl.BlockSpec((HID, LAT), lambda g, i: (0, 0)),
        ],
        out_specs=pl.BlockSpec((1, blk, LAT), lambda g, i: (g, i, 0)),
        out_shape=jax.ShapeDtypeStruct((2, N, LAT), jnp.float32),
    )(y1pre, w1p, b1.reshape(1, HID), wmu)


def _head_body(h_ref, r1_ref, rb1_ref, r2_ref, rb2_ref, r3_ref, rb3_ref,
               o_ref):
    h = jnp.maximum(
        jnp.dot(h_ref[...], r1_ref[...], preferred_element_type=jnp.float32)
        + rb1_ref[...], 0.0)
    h = jnp.maximum(
        jnp.dot(h, r2_ref[...], preferred_element_type=jnp.float32)
        + rb2_ref[...], 0.0)
    o_ref[...] = (
        jnp.dot(h, r3_ref[...], preferred_element_type=jnp.float32)
        + rb3_ref[...])


def _head_call(hcat, R1, rb1, R2, rb2, R3, rb3):
    return pl.pallas_call(
        _head_body,
        out_shape=jax.ShapeDtypeStruct((G, 2), jnp.float32),
    )(hcat, R1, rb1.reshape(1, -1), R2, rb2.reshape(1, -1), R3,
      rb3.reshape(1, -1))


def _pad_to(a, n, val):
    return jnp.pad(a, ((0, n - a.shape[0]),), constant_values=val)


def kernel(xA, edge_indexA, batchA, xB, edge_indexB, batchB, W1, b1, Wmu, bmu,
           Wstd, bstd, R1, rb1, R2, rb2, R3, rb3):
    src1 = jnp.stack([_pad_to(edge_indexA[0], EP, 0),
                      _pad_to(edge_indexB[0], EP, 0)])
    dst1 = jnp.stack([_pad_to(edge_indexA[1], EP, N),
                      _pad_to(edge_indexB[1], EP, N)])
    bat1 = jnp.stack([_pad_to(batchA, NPAD, G), _pad_to(batchB, NPAD, G)])

    degs = _deg_kernel(dst1)[:, :N]
    dinv = lax.rsqrt(degs + 1.0)

    x = jnp.stack([xA, xB])
    xp = jnp.pad(x * dinv[:, :, None], ((0, 0), (0, 0), (0, DP - D)))
    tabs1 = jnp.stack([xp[:, :, :32], xp[:, :, 32:]], axis=1)
    s1 = _conv1_kernel(tabs1, src1, dst1)
    s1f = jnp.concatenate([s1[:, 0, :N], s1[:, 1, :N]], axis=-1)
    y1pre = dinv[:, :, None] * (s1f + xp)

    w1p = jnp.pad(W1, ((0, DP - D), (0, 0)))
    u = _mid_call(y1pre, w1p, b1, Wmu)
    up = dinv[:, :, None] * u

    s2 = _conv2_kernel(up, src1, dst1)
    v = dinv[:, :, None] * (s2[:, :N] + up)
    vext = jnp.concatenate([
        jnp.pad(v, ((0, 0), (0, NPAD - N), (0, 0))),
        jnp.pad(jnp.ones((2, N, 1), jnp.float32),
                ((0, 0), (0, NPAD - N), (0, 0))),
        jnp.zeros((2, NPAD, 15), jnp.float32),
    ], axis=-1)
    pooled = _pool_kernel(vext, bat1)
    sums = pooled[:, :G, :LAT]
    cnt = pooled[:, :G, LAT]
    z = (sums + cnt[:, :, None] * bmu) / jnp.maximum(cnt, 1.0)[:, :, None]

    hcat = jnp.concatenate([z[0], z[1]], axis=1)
    return _head_call(hcat, R1, rb1, R2, rb2, R3, rb3)


# speedup vs baseline: 19.1560x; 1.1423x over previous
import functools

import jax
import jax.numpy as jnp
from jax import lax
from jax.experimental import pallas as pl
from jax.experimental.pallas import tpu as pltpu
from jax.experimental.pallas import tpu_sc as plsc

N = 50000
E = 800000
D = 58
DP = 64
HID = 128
LAT = 32
G = 1024

NC = 2
NS = 16
EP = 819200
EPT = EP // NS
NACC = 51200
RPT = NACC // NS
GACC = 1040
GPT = GACC // NS
NPAD = 51200
NPT = NPAD // NS

LS = 256
NB2 = 2
ECH = 2048
DCH = 2048
PCH = 640

_mesh = plsc.VectorSubcoreMesh(
    core_axis_name="c", subcore_axis_name="s", num_cores=NC, num_subcores=NS)
_sc_params = pltpu.CompilerParams(use_tc_tiling_on_sc=False)


def _zero_vmem(ref, n_f32):
    z = jnp.zeros((16,), jnp.float32)
    if ref.ndim == 1:
        for i in range(n_f32 // 16):
            ref[pl.ds(i * 16, 16)] = z
    else:
        rows, cols = ref.shape
        for r in range(rows):
            for j in range(cols // 16):
                ref[r, pl.ds(j * 16, 16)] = z


def _deg_body(dst1, deg_out, acc, idx, ones_v, zb, sem):
    c = lax.axis_index("c")
    s = lax.axis_index("s")
    base = pl.multiple_of(s * RPT, 8)
    _zero_vmem(zb, RPT)
    for i in range(DCH // 16):
        ones_v[pl.ds(i * 16, 16)] = jnp.ones((16,), jnp.float32)
    pltpu.sync_copy(zb, acc.at[pl.ds(base, RPT)])
    plsc.subcore_barrier()
    pltpu.sync_copy(dst1.at[c, pl.ds(s * EPT, EPT)], idx)
    for b in range(EPT // DCH):
        pltpu.async_copy(ones_v, acc.at[idx.at[pl.ds(b * DCH, DCH)]], sem,
                         add=True).wait()
    plsc.subcore_barrier()
    pltpu.sync_copy(acc.at[pl.ds(base, RPT)], zb)
    pltpu.sync_copy(zb, deg_out.at[c, pl.ds(base, RPT)])


_deg_kernel = functools.partial(
    pl.kernel,
    out_type=jax.ShapeDtypeStruct((NC, NACC), jnp.float32),
    mesh=_mesh,
    compiler_params=_sc_params,
    scratch_types=[
        pltpu.VMEM_SHARED((NACC,), jnp.float32),
        pltpu.VMEM((EPT,), jnp.int32),
        pltpu.VMEM((DCH,), jnp.float32),
        pltpu.VMEM((RPT,), jnp.float32),
        pltpu.SemaphoreType.DMA,
    ],
)(_deg_body)


def _edge_pass(tab, src1, dst1, out, acc, idx_s, idx_d, rbuf, zrow,
               semg, sems, s):
    w = zrow.shape[1]
    base = pl.multiple_of(s * RPT, 8)
    _zero_vmem(zrow, 128 * w)
    for i in range(RPT // 128):
        pltpu.sync_copy(zrow, acc.at[pl.ds(base + i * 128, 128)])
    plsc.subcore_barrier()
    for step in range(EPT // ECH):
        e0 = s * EPT + step * ECH
        pltpu.sync_copy(src1.at[pl.ds(e0, ECH)], idx_s)
        pltpu.sync_copy(dst1.at[pl.ds(e0, ECH)], idx_d)

        def batch(b, _):
            r = pl.multiple_of(b * (NB2 * LS), LS)
            gd = [
                pltpu.async_copy(tab.at[idx_s.at[pl.ds(r + j * LS, LS)]],
                                 rbuf.at[j], semg)
                for j in range(NB2)
            ]
            for d in gd:
                d.wait()
            sd = [
                pltpu.async_copy(rbuf.at[j],
                                 acc.at[idx_d.at[pl.ds(r + j * LS, LS)]],
                                 sems, add=True)
                for j in range(NB2)
            ]
            for d in sd:
                d.wait()
            return _

        lax.fori_loop(0, ECH // (NB2 * LS), batch, None)
    plsc.subcore_barrier()
    for i in range(RPT // 128):
        pltpu.sync_copy(acc.at[pl.ds(base + i * 128, 128)], zrow)
        pltpu.sync_copy(zrow, out.at[pl.ds(base + i * 128, 128)])


_conv_scratch = [
    pltpu.VMEM_SHARED((NACC, 32), jnp.float32),
    pltpu.VMEM((ECH,), jnp.int32),
    pltpu.VMEM((ECH,), jnp.int32),
    pltpu.VMEM((NB2, LS, 32), jnp.float32),
    pltpu.VMEM((128, 32), jnp.float32),
    pltpu.SemaphoreType.DMA,
    pltpu.SemaphoreType.DMA,
]


def _conv1_body(tabs, src1, dst1, s1, acc, idx_s, idx_d, rbuf, zrow,
                semg, sems):
    c = lax.axis_index("c")
    s = lax.axis_index("s")
    for g in range(2):
        _edge_pass(tabs.at[g, c], src1.at[g], dst1.at[g], s1.at[g, c],
                   acc, idx_s, idx_d, rbuf, zrow, semg, sems, s)
        if g == 0:
            plsc.subcore_barrier()


_conv1_kernel = functools.partial(
    pl.kernel,
    out_type=jax.ShapeDtypeStruct((2, NC, NACC, 32), jnp.float32),
    mesh=_mesh,
    compiler_params=_sc_params,
    scratch_types=_conv_scratch,
)(_conv1_body)


def _conv2_body(tabs, src1, dst1, s2, tabsp, acc, sbuf, idx_s, idx_d, rbuf,
                zrow, semg, sems):
    c = lax.axis_index("c")
    s = lax.axis_index("s")
    nrs = N // NS
    for p in range(2):
        for i in range(nrs // 125):
            r0 = s * nrs + i * 125
            pltpu.sync_copy(tabs.at[c, p, pl.ds(r0, 125)], sbuf)
            pltpu.sync_copy(sbuf, tabsp.at[pl.ds(r0, 125)])
        _edge_pass(tabsp, src1.at[c], dst1.at[c], s2.at[c, p],
                   acc, idx_s, idx_d, rbuf, zrow, semg, sems, s)
        if p == 0:
            plsc.subcore_barrier()


_conv2_kernel = functools.partial(
    pl.kernel,
    out_type=jax.ShapeDtypeStruct((NC, 2, NACC, 16), jnp.float32),
    mesh=_mesh,
    compiler_params=_sc_params,
    scratch_types=[
        pltpu.VMEM_SHARED((N, 16), jnp.float32),
        pltpu.VMEM_SHARED((NACC, 16), jnp.float32),
        pltpu.VMEM((125, 16), jnp.float32),
        pltpu.VMEM((ECH,), jnp.int32),
        pltpu.VMEM((ECH,), jnp.int32),
        pltpu.VMEM((NB2, LS, 16), jnp.float32),
        pltpu.VMEM((128, 16), jnp.float32),
        pltpu.SemaphoreType.DMA,
        pltpu.SemaphoreType.DMA,
    ],
)(_conv2_body)


def _pool_body(vext, bat1, pool_out, acc, idx, vbuf, zrow, semg, sems):
    c = lax.axis_index("c")
    s = lax.axis_index("s")
    _zero_vmem(zrow, GPT * 48)
    pltpu.sync_copy(zrow, acc.at[pl.ds(s * GPT, GPT)])
    plsc.subcore_barrier()
    pltpu.sync_copy(bat1.at[c, pl.ds(s * NPT, NPT)], idx)
    for b in range(NPT // PCH):
        j = b % 2
        pltpu.async_copy(vext.at[c, pl.ds(s * NPT + b * PCH, PCH)],
                         vbuf.at[j], semg).wait()
        pltpu.async_copy(vbuf.at[j], acc.at[idx.at[pl.ds(b * PCH, PCH)]],
                         sems, add=True).wait()
    plsc.subcore_barrier()
    pltpu.sync_copy(acc.at[pl.ds(s * GPT, GPT)], zrow)
    pltpu.sync_copy(zrow, pool_out.at[c, pl.ds(s * GPT, GPT)])


_pool_kernel = functools.partial(
    pl.kernel,
    out_type=jax.ShapeDtypeStruct((NC, GACC, 48), jnp.float32),
    mesh=_mesh,
    compiler_params=_sc_params,
    scratch_types=[
        pltpu.VMEM_SHARED((GACC, 48), jnp.float32),
        pltpu.VMEM((NPT,), jnp.int32),
        pltpu.VMEM((2, PCH, 48), jnp.float32),
        pltpu.VMEM((GPT, 48), jnp.float32),
        pltpu.SemaphoreType.DMA,
        pltpu.SemaphoreType.DMA,
    ],
)(_pool_body)


def _mid_body(y_ref, w1_ref, b1_ref, wmu_ref, u_ref):
    y = y_ref[0]
    h = jnp.maximum(
        jnp.dot(y, w1_ref[...], preferred_element_type=jnp.float32)
        + b1_ref[...], 0.0)
    u_ref[0] = jnp.dot(h, wmu_ref[...], preferred_element_type=jnp.float32)


def _mid_call(y1pre, w1p, b1, wmu):
    blk = 2000
    return pl.pallas_call(
        _mid_body,
        grid=(2, N // blk),
        in_specs=[
            pl.BlockSpec((1, blk, DP), lambda g, i: (g, i, 0)),
            pl.BlockSpec((DP, HID), lambda g, i: (0, 0)),
            pl.BlockSpec((1, HID), lambda g, i: (0, 0)),
            pl.BlockSpec((HID, LAT), lambda g, i: (0, 0)),
        ],
        out_specs=pl.BlockSpec((1, blk, LAT), lambda g, i: (g, i, 0)),
        out_shape=jax.ShapeDtypeStruct((2, N, LAT), jnp.float32),
    )(y1pre, w1p, b1.reshape(1, HID), wmu)


def _head_body(h_ref, r1_ref, rb1_ref, r2_ref, rb2_ref, r3_ref, rb3_ref,
               o_ref):
    h = jnp.maximum(
        jnp.dot(h_ref[...], r1_ref[...], preferred_element_type=jnp.float32)
        + rb1_ref[...], 0.0)
    h = jnp.maximum(
        jnp.dot(h, r2_ref[...], preferred_element_type=jnp.float32)
        + rb2_ref[...], 0.0)
    o_ref[...] = (
        jnp.dot(h, r3_ref[...], preferred_element_type=jnp.float32)
        + rb3_ref[...])


def _head_call(hcat, R1, rb1, R2, rb2, R3, rb3):
    return pl.pallas_call(
        _head_body,
        out_shape=jax.ShapeDtypeStruct((G, 2), jnp.float32),
    )(hcat, R1, rb1.reshape(1, -1), R2, rb2.reshape(1, -1), R3,
      rb3.reshape(1, -1))


def _pad_to(a, n, lo, hi):
    pad = lo + jnp.arange(n - a.shape[0], dtype=a.dtype) % (hi - lo)
    return jnp.concatenate([a, pad])


def kernel(xA, edge_indexA, batchA, xB, edge_indexB, batchB, W1, b1, Wmu, bmu,
           Wstd, bstd, R1, rb1, R2, rb2, R3, rb3):
    src1 = jnp.stack([_pad_to(edge_indexA[0], EP, 0, N),
                      _pad_to(edge_indexB[0], EP, 0, N)])
    dst1 = jnp.stack([_pad_to(edge_indexA[1], EP, N, NACC),
                      _pad_to(edge_indexB[1], EP, N, NACC)])
    bat1 = jnp.stack([_pad_to(batchA, NPAD, G, GACC),
                      _pad_to(batchB, NPAD, G, GACC)])

    degs = _deg_kernel(dst1)[:, :N]
    dinv = lax.rsqrt(degs + 1.0)

    x = jnp.stack([xA, xB])
    xp = jnp.pad(x * dinv[:, :, None], ((0, 0), (0, 0), (0, DP - D)))
    tabs1 = jnp.stack([xp[:, :, :32], xp[:, :, 32:]], axis=1)
    s1 = _conv1_kernel(tabs1, src1, dst1)
    s1f = jnp.concatenate([s1[:, 0, :N], s1[:, 1, :N]], axis=-1)
    y1pre = dinv[:, :, None] * (s1f + xp)

    w1p = jnp.pad(W1, ((0, DP - D), (0, 0)))
    u = _mid_call(y1pre, w1p, b1, Wmu)
    up = dinv[:, :, None] * u

    uph = jnp.stack([up[:, :, :16], up[:, :, 16:]], axis=1)
    s2h = _conv2_kernel(uph, src1, dst1)
    s2 = jnp.concatenate([s2h[:, 0, :N], s2h[:, 1, :N]], axis=-1)
    v = dinv[:, :, None] * (s2 + up)
    vext = jnp.concatenate([
        jnp.pad(v, ((0, 0), (0, NPAD - N), (0, 0))),
        jnp.pad(jnp.ones((2, N, 1), jnp.float32),
                ((0, 0), (0, NPAD - N), (0, 0))),
        jnp.zeros((2, NPAD, 15), jnp.float32),
    ], axis=-1)
    pooled = _pool_kernel(vext, bat1)
    sums = pooled[:, :G, :LAT]
    cnt = pooled[:, :G, LAT]
    z = (sums + cnt[:, :, None] * bmu) / jnp.maximum(cnt, 1.0)[:, :, None]

    hcat = jnp.concatenate([z[0], z[1]], axis=1)
    return _head_call(hcat, R1, rb1, R2, rb2, R3, rb3)
